# baseline (speedup 1.0000x reference)
import jax
import jax.numpy as jnp
from jax import lax
from jax.experimental import pallas as pl
from jax.experimental.pallas import tpu as pltpu
from jax.experimental.pallas import tpu_sc as plsc

_R, _N = 128, 32768
_SCR = 32
_NSL = 8
_SLC = _N // _NSL
_CHUNK = 2048
_NCHUNKS = _SLC // _CHUNK
_BR = 32
_BIG = 2**30


def _sc_argmax_body(probs_hbm, maxes_hbm, idxs_hbm, buf_v, res_v, tmp_v, sem0, sem1):
    c = lax.axis_index("c")
    s = lax.axis_index("s")
    wid = c * 16 + s
    g = wid // _NSL
    sl = wid % _NSL
    row0 = g * 8
    col0 = sl * _SLC
    lanes = lax.iota(jnp.int32, 16)
    sems = [sem0, sem1]

    def chunk_src(ci):
        return probs_hbm.at[pl.ds(row0, 8), pl.ds(col0 + ci * _CHUNK, _CHUNK)]

    cp = pltpu.async_copy(chunk_src(0), buf_v.at[0], sems[0])
    maxv = [jnp.full((16,), -1.0, jnp.float32) for _ in range(8)]
    maxj = [jnp.zeros((16,), jnp.int32) for _ in range(8)]
    for ci in range(_NCHUNKS):
        b = ci % 2
        cp_next = None
        if ci + 1 < _NCHUNKS:
            cp_next = pltpu.async_copy(
                chunk_src(ci + 1), buf_v.at[(ci + 1) % 2], sems[(ci + 1) % 2]
            )
        cp.wait()

        def body(j, state):
            out = []
            jg = ci * (_CHUNK // 16) + j
            for r in range(8):
                m, mj = state[2 * r], state[2 * r + 1]
                x = buf_v[b, r, pl.ds(j * 16, 16)]
                gt = x > m
                out.append(jnp.where(gt, x, m))
                out.append(jnp.where(gt, jg, mj))
            return tuple(out)

        state = []
        for r in range(8):
            state += [maxv[r], maxj[r]]
        state = lax.fori_loop(0, _CHUNK // 16, body, tuple(state))
        for r in range(8):
            maxv[r], maxj[r] = state[2 * r], state[2 * r + 1]
        cp = cp_next

    gmax_vec = jnp.zeros((16,), jnp.float32)
    gidx_vec = jnp.zeros((16,), jnp.int32)
    for r in range(8):
        gmax = jnp.max(maxv[r], axis=0)
        ei = maxj[r] * 16 + lanes + col0
        cand = jnp.where(maxv[r] == gmax, ei, _BIG)
        gidx = jnp.min(cand, axis=0)
        gmax_vec = jnp.where(lanes == r, gmax, gmax_vec)
        gidx_vec = jnp.where(lanes == r, gidx, gidx_vec)

    res_v[pl.ds(0, 16)] = gmax_vec
    tmp_v[pl.ds(0, 16)] = gidx_vec
    pltpu.sync_copy(res_v, maxes_hbm.at[pl.ds(wid * 16, 16)])
    pltpu.sync_copy(tmp_v, idxs_hbm.at[pl.ds(wid * 16, 16)])


_sc_argmax = pl.kernel(
    _sc_argmax_body,
    out_type=(
        jax.ShapeDtypeStruct((512,), jnp.float32),
        jax.ShapeDtypeStruct((512,), jnp.int32),
    ),
    mesh=plsc.VectorSubcoreMesh(core_axis_name="c", subcore_axis_name="s"),
    scratch_types=[
        pltpu.VMEM((2, 8, _CHUNK), jnp.float32),
        pltpu.VMEM((16,), jnp.float32),
        pltpu.VMEM((16,), jnp.int32),
        pltpu.SemaphoreType.DMA,
        pltpu.SemaphoreType.DMA,
    ],
    compiler_params=pltpu.CompilerParams(needs_layout_passes=False),
)


def _argmax_body(x_ref, idx_ref):
    x = x_ref[...]
    m = jnp.max(x, axis=1, keepdims=True)
    iota = lax.broadcasted_iota(jnp.int32, x.shape, 1)
    idx_ref[...] = jnp.min(jnp.where(x == m, iota, _N), axis=1, keepdims=True)


def _argmax_tc(probs):
    return pl.pallas_call(
        _argmax_body,
        grid=((_R - _SCR) // _BR,),
        in_specs=[pl.BlockSpec((_BR, _N), lambda i: (i + 1, 0))],
        out_specs=pl.BlockSpec((_BR, 1), lambda i: (i, 0)),
        out_shape=jax.ShapeDtypeStruct((_R - _SCR, 1), jnp.int32),
    )(probs)


def kernel(probs):
    maxes, idxs = _sc_argmax(probs)
    idx_tc = _argmax_tc(probs)
    m8 = maxes.reshape(4, _NSL, 16)[:, :, :8]
    i8 = idxs.reshape(4, _NSL, 16)[:, :, :8]
    best = m8.max(axis=1)
    idx_sc = jnp.min(jnp.where(m8 == best[:, None, :], i8, _BIG), axis=1)
    idx_all = jnp.concatenate([idx_sc.reshape(_SCR), idx_tc.reshape(_R - _SCR)])
    return jnp.arange(_N, dtype=jnp.int32)[None, :] == idx_all[:, None]


# speedup vs baseline: 2.3178x; 2.3178x over previous
import jax
import jax.numpy as jnp
from jax import lax
from jax.experimental import pallas as pl

_R, _N = 128, 32768
_BR = 64


def _argmax_body(x_ref, idx_ref):
    x = x_ref[...]
    m = jnp.max(x, axis=1, keepdims=True)
    iota = lax.broadcasted_iota(jnp.int32, x.shape, 1)
    idx_ref[...] = jnp.min(jnp.where(x == m, iota, _N), axis=1, keepdims=True)


def _argmax_tc(probs):
    return pl.pallas_call(
        _argmax_body,
        grid=(_R // _BR,),
        in_specs=[pl.BlockSpec((_BR, _N), lambda i: (i, 0))],
        out_specs=pl.BlockSpec((_BR, 1), lambda i: (i, 0)),
        out_shape=jax.ShapeDtypeStruct((_R, 1), jnp.int32),
    )(probs)


def kernel(probs):
    idx = _argmax_tc(probs)
    onehot = jnp.arange(_N, dtype=jnp.int32)[None, :] == idx
    return onehot
